# Initial kernel scaffold; baseline (speedup 1.0000x reference)
import jax
import jax.numpy as jnp
from jax.experimental import pallas as pl


def kernel(x, edge_index, W1, a_src1, a_dst1, b1, gamma1, beta1, W2, a_src2, a_dst2, b2):
    raise NotImplementedError("write your pallas kernel here")


# speedup vs baseline: 33.9546x; 33.9546x over previous
import functools

import jax
import jax.numpy as jnp
from jax import lax
from jax.experimental import pallas as pl
from jax.experimental.pallas import tpu as pltpu
from jax.experimental.pallas import tpu_sc as plsc

_N = 10000
_E = 320000
_NP = 10240
_CC = 80
_ROWS_PER_TILE = _NP // 16
_NCH1 = 157
_EP1 = 16 * _NCH1 * 128
_NCH2 = 79
_EP2 = 32 * _NCH2 * 128

_f32 = jnp.float32


def _make_edge_kernel(nwork, nch, row_off):
    mesh = plsc.VectorSubcoreMesh(core_axis_name="c", subcore_axis_name="s")

    @functools.partial(
        pl.kernel,
        mesh=mesh,
        compiler_params=pltpu.CompilerParams(use_tc_tiling_on_sc=False),
        out_type=jax.ShapeDtypeStruct((2, _NP, _CC), _f32),
        scratch_types=[
            pltpu.VMEM((128,), jnp.int32),
            pltpu.VMEM((128,), jnp.int32),
            pltpu.VMEM((128,), jnp.int32),
            pltpu.VMEM((128, _CC), _f32),
            pltpu.VMEM((128, _CC), _f32),
            pltpu.VMEM((128, _CC), _f32),
            pltpu.VMEM((128, _CC), _f32),
            pltpu.VMEM_SHARED((_NP, _CC), _f32),
            pltpu.SemaphoreType.DMA,
            pltpu.SemaphoreType.DMA,
            pltpu.SemaphoreType.DMA,
        ],
    )
    def ek(h_hbm, as_hbm, ad_hbm, src_hbm, dst_hbm, acc_hbm,
           si_v, di_v, di2, asr, adr, h_v, msg, acc_s, sm1, sm2, sm3):
        cid = lax.axis_index("c")
        sid = lax.axis_index("s")
        gw = sid if nwork == 16 else cid * 16 + sid
        off = cid * row_off

        zero16 = jnp.zeros((16,), _f32)

        def zrow(j, c):
            for k in range(_CC // 16):
                msg[j, pl.ds(k * 16, 16)] = zero16
            return c

        lax.fori_loop(0, 128, zrow, 0)
        for k in range(_ROWS_PER_TILE // 128):
            pltpu.sync_copy(msg, acc_s.at[pl.ds(sid * _ROWS_PER_TILE + k * 128, 128)])
        plsc.subcore_barrier()

        def chunk(i, c):
            pltpu.sync_copy(src_hbm.at[gw, i], si_v)
            pltpu.sync_copy(dst_hbm.at[gw, i], di_v)
            for k in range(8):
                sl = pl.ds(k * 16, 16)
                si_v[sl] = si_v[sl] + off
                di2[sl] = di_v[sl] + off
            cp1 = pltpu.async_copy(as_hbm.at[si_v], asr, sm1)
            cp2 = pltpu.async_copy(ad_hbm.at[di2], adr, sm2)
            cp3 = pltpu.async_copy(h_hbm.at[si_v], h_v, sm3)
            cp1.wait()
            cp2.wait()

            def edge(j, cc):
                for k in range(_CC // 16):
                    sl = pl.ds(k * 16, 16)
                    a = asr[j, sl] + adr[j, sl]
                    e = jnp.maximum(a, 0.2 * a)
                    msg[j, sl] = jnp.exp(e)
                return cc

            lax.fori_loop(0, 128, edge, 0)
            cp3.wait()

            def wmul(j, cc):
                for k in range(_CC // 16):
                    sl = pl.ds(k * 16, 16)
                    msg[j, sl] = msg[j, sl] * h_v[j, sl]
                return cc

            lax.fori_loop(0, 128, wmul, 0)
            pltpu.sync_copy(msg, acc_s.at[di_v], add=True)
            return c

        lax.fori_loop(0, nch, chunk, 0)
        plsc.subcore_barrier()
        pltpu.sync_copy(acc_s.at[pl.ds(sid * _ROWS_PER_TILE, _ROWS_PER_TILE)],
                        acc_hbm.at[cid, pl.ds(sid * _ROWS_PER_TILE, _ROWS_PER_TILE)])

    return ek


_edge_kernel_1 = _make_edge_kernel(16, _NCH1, _NP)
_edge_kernel_2 = _make_edge_kernel(32, _NCH2, 0)


def _tc_a_body(x_ref, w_ref, s_ref, d_ref, h_out, as_out, ad_out):
    h = jnp.dot(x_ref[...], w_ref[...], preferred_element_type=_f32)
    ones16 = jnp.ones((_NP, 16), _f32)
    h_out[0:_NP] = jnp.concatenate([h[:, 0:64], ones16], axis=1)
    h_out[_NP:2 * _NP] = jnp.concatenate([h[:, 64:128], ones16], axis=1)
    asf = jnp.dot(h, s_ref[...], preferred_element_type=_f32)
    adf = jnp.dot(h, d_ref[...], preferred_element_type=_f32)
    as_out[0:_NP] = asf[:, 0:_CC]
    as_out[_NP:2 * _NP] = asf[:, _CC:2 * _CC]
    ad_out[0:_NP] = adf[:, 0:_CC]
    ad_out[_NP:2 * _NP] = adf[:, _CC:2 * _CC]


def _tc_b_body(acc_ref, g_ref, bt_ref, b1_ref, w2_ref, s2_ref, d2_ref, eh_ref,
               h2_out, as2_out, ad2_out):
    m0 = acc_ref[0, :, 0:64]
    m1 = acc_ref[1, :, 0:64]
    d0 = jnp.dot(acc_ref[0, :, 64:80], eh_ref[...], preferred_element_type=_f32) + 1e-16
    d1 = jnp.dot(acc_ref[1, :, 64:80], eh_ref[...], preferred_element_type=_f32) + 1e-16
    h_gat = jnp.concatenate([m0 / d0, m1 / d1], axis=1) + b1_ref[...]
    m = jnp.mean(h_gat[:_N], axis=0, keepdims=True)
    xc = h_gat - m
    var = jnp.mean(jnp.square(xc[:_N]), axis=0, keepdims=True)
    hbn = xc / jnp.sqrt(var + 1e-5) * g_ref[...] + bt_ref[...]
    hr = jnp.maximum(hbn, 0.0)
    h2 = jnp.dot(hr, w2_ref[...], preferred_element_type=_f32)
    h2_out[...] = jnp.concatenate([h2, jnp.ones((_NP, 16), _f32)], axis=1)
    as2_out[...] = jnp.dot(h2, s2_ref[...], preferred_element_type=_f32)
    ad2_out[...] = jnp.dot(h2, d2_ref[...], preferred_element_type=_f32)


def _tc_c_body(acc_ref, e2_ref, b2_ref, out_ref):
    s = acc_ref[0] + acc_ref[1]
    s = s[:_N]
    den = jnp.dot(s[:, 64:80], e2_ref[...], preferred_element_type=_f32) + 1e-16
    out_ref[...] = s[:, 0:64] / den + b2_ref[...]


def kernel(x, edge_index, W1, a_src1, a_dst1, b1, gamma1, beta1, W2, a_src2, a_dst2, b2):
    xp = jnp.zeros((_NP, 128), _f32).at[:_N].set(x)

    src = edge_index[0].astype(jnp.int32)
    dst = edge_index[1].astype(jnp.int32)
    pad1 = jnp.full((_EP1 - _E,), _N, jnp.int32)
    src1 = jnp.concatenate([src, pad1]).reshape(16, _NCH1, 128)
    dst1 = jnp.concatenate([dst, pad1]).reshape(16, _NCH1, 128)
    pad2 = jnp.full((_EP2 - _E,), _N, jnp.int32)
    src2 = jnp.concatenate([src, pad2]).reshape(32, _NCH2, 128)
    dst2 = jnp.concatenate([dst, pad2]).reshape(32, _NCH2, 128)

    q = jnp.arange(2 * _CC)
    qm = q % _CC
    colhead = 4 * (q // _CC) + jnp.where(qm < 64, qm // 16, qm - 64)
    valid = qm < 68
    chead = jnp.arange(128) // 16
    mask1 = ((colhead[None, :] == chead[:, None]) & valid[None, :]).astype(_f32)
    S1 = a_src1.reshape(128, 1) * mask1
    D1 = a_dst1.reshape(128, 1) * mask1

    mask2 = (jnp.arange(_CC)[None, :] <= 64).astype(_f32) * jnp.ones((64, 1), _f32)
    S2 = a_src2.reshape(64, 1) * mask2
    D2 = a_dst2.reshape(64, 1) * mask2

    Eh = ((jnp.arange(64)[None, :] // 16) == jnp.arange(16)[:, None]).astype(_f32)
    E2 = (jnp.arange(16)[:, None] == 0).astype(_f32) * jnp.ones((1, 64), _f32)

    b1r = b1.reshape(1, 128)
    g1r = gamma1.reshape(1, 128)
    bt1r = beta1.reshape(1, 128)
    b2r = b2.reshape(1, 64)

    h1aug, asf1, adf1 = pl.pallas_call(
        _tc_a_body,
        out_shape=(
            jax.ShapeDtypeStruct((2 * _NP, _CC), _f32),
            jax.ShapeDtypeStruct((2 * _NP, _CC), _f32),
            jax.ShapeDtypeStruct((2 * _NP, _CC), _f32),
        ),
    )(xp, W1, S1, D1)

    acc1 = _edge_kernel_1(h1aug, asf1, adf1, src1, dst1)

    h2aug, asf2, adf2 = pl.pallas_call(
        _tc_b_body,
        out_shape=(
            jax.ShapeDtypeStruct((_NP, _CC), _f32),
            jax.ShapeDtypeStruct((_NP, _CC), _f32),
            jax.ShapeDtypeStruct((_NP, _CC), _f32),
        ),
    )(acc1, g1r, bt1r, b1r, W2, S2, D2, Eh)

    acc2 = _edge_kernel_2(h2aug, asf2, adf2, src2, dst2)

    out = pl.pallas_call(
        _tc_c_body,
        out_shape=jax.ShapeDtypeStruct((_N, 64), _f32),
    )(acc2, E2, b2r)
    return out
